# Initial kernel scaffold; baseline (speedup 1.0000x reference)
import jax
import jax.numpy as jnp
from jax.experimental import pallas as pl


def kernel(images, feat_p3, feat_p4, feat_p5, conv_w, conv_b, cls_w, cls_b, bbox_w, bbox_b):
    raise NotImplementedError("write your pallas kernel here")


# speedup vs baseline: 1.0522x; 1.0522x over previous
import functools
import math

import jax
import jax.numpy as jnp
import numpy as np
from jax.experimental import pallas as pl
from jax.experimental.pallas import tpu as pltpu

_SIZES = [32, 64, 128, 256, 512]
_RATIOS = [0.5, 1.0, 2.0]

_LEVELS = [(64, 64), (32, 32), (16, 16)]


@functools.lru_cache(maxsize=None)
def _anchors_const(img_h, grids):
    per_all = []
    for grid in grids:
        scale = img_h / grid
        steps = (np.arange(grid, dtype=np.float32)
                 * np.float32(scale)).astype(np.float32)
        x, y = np.meshgrid(steps, steps, indexing='ij')
        for s in _SIZES:
            for r in _RATIOS:
                rs = math.sqrt(r)
                aw = np.full((grid, grid), np.float32(s * rs), dtype=np.float32)
                ah = np.full((grid, grid), np.float32(s / rs), dtype=np.float32)
                a = np.stack((x, y, aw, ah)).transpose(1, 2, 0).reshape(-1, 4)
                per_all.append(a)
    return np.concatenate(per_all, axis=0)


def _rpn_head_kernel(x3, x4, x5, w9, cb, clw, clb, bbw, bbb,
                     lo3, bb3, lo4, bb4, lo5, bb5):
    outs = ((x3, lo3, bb3), (x4, lo4, bb4), (x5, lo5, bb5))
    for (h, w), (x, lo, bb) in zip(_LEVELS, outs):
        wp = w + 2
        np_ = (h + 2) * (w + 2)
        acc = jnp.zeros((256, np_), dtype=jnp.float32)
        for k in range(9):
            off = (k // 3) * wp + (k % 3)
            acc += jnp.dot(w9[k], x[:, off:off + np_],
                           preferred_element_type=jnp.float32)
        t = jnp.maximum(acc + cb[...], 0.0)
        lo[...] = jnp.dot(clw[...], t, preferred_element_type=jnp.float32) + clb[...]
        bb[...] = jnp.dot(bbw[...], t, preferred_element_type=jnp.float32) + bbb[...]


def _prep_level(feat, h, w):
    wp = w + 2
    xp = jnp.pad(feat[0], ((0, 0), (1, 1), (1, 1)))
    xp = xp.reshape(256, (h + 2) * wp)
    return jnp.pad(xp, ((0, 0), (wp + 1, wp + 1)))


def _extract(full, h, w, c):
    return full.reshape(c, h + 2, w + 2)[:, 1:h + 1, 1:w + 1].reshape(1, -1)


def kernel(images, feat_p3, feat_p4, feat_p5, conv_w, conv_b,
           cls_w, cls_b, bbox_w, bbox_b):
    feats = [feat_p3, feat_p4, feat_p5]
    xs = [_prep_level(f, h, w) for f, (h, w) in zip(feats, _LEVELS)]

    w9 = conv_w.transpose(2, 3, 0, 1).reshape(9, 256, 256)
    cb = conv_b.reshape(256, 1)
    clw = cls_w.reshape(15, 256)
    clb = cls_b.reshape(15, 1)
    bbw = bbox_w.reshape(60, 256)
    bbb = bbox_b.reshape(60, 1)

    out_shapes = []
    for h, w in _LEVELS:
        np_ = (h + 2) * (w + 2)
        out_shapes.append(jax.ShapeDtypeStruct((15, np_), jnp.float32))
        out_shapes.append(jax.ShapeDtypeStruct((60, np_), jnp.float32))

    lo3, bb3, lo4, bb4, lo5, bb5 = pl.pallas_call(
        _rpn_head_kernel,
        out_shape=tuple(out_shapes),
    )(xs[0], xs[1], xs[2], w9, cb, clw, clb, bbw, bbb)

    pieces = []
    for (h, w), lo, bb in zip(_LEVELS, (lo3, lo4, lo5), (bb3, bb4, bb5)):
        pieces.append(_extract(lo, h, w, 15))
        pieces.append(_extract(bb, h, w, 60))
    flat = jnp.concatenate(pieces, axis=1)

    anchors = jnp.asarray(
        _anchors_const(images.shape[-2], tuple(h for h, _ in _LEVELS)))
    return (flat, anchors)


# speedup vs baseline: 1.2050x; 1.1452x over previous
import functools
import math

import jax
import jax.numpy as jnp
import numpy as np
from jax.experimental import pallas as pl
from jax.experimental.pallas import tpu as pltpu

_SIZES = [32, 64, 128, 256, 512]
_RATIOS = [0.5, 1.0, 2.0]

_LEVELS = [(64, 64), (32, 32), (16, 16)]


@functools.lru_cache(maxsize=None)
def _anchors_const(img_h, grids):
    per_all = []
    for grid in grids:
        scale = img_h / grid
        steps = (np.arange(grid, dtype=np.float32)
                 * np.float32(scale)).astype(np.float32)
        x, y = np.meshgrid(steps, steps, indexing='ij')
        for s in _SIZES:
            for r in _RATIOS:
                rs = math.sqrt(r)
                aw = np.full((grid, grid), np.float32(s * rs), dtype=np.float32)
                ah = np.full((grid, grid), np.float32(s / rs), dtype=np.float32)
                a = np.stack((x, y, aw, ah)).transpose(1, 2, 0).reshape(-1, 4)
                per_all.append(a)
    return np.concatenate(per_all, axis=0)


def _rpn_head_kernel(x3, x4, x5, w9, cb, clw, clb, bbw, bbb,
                     lo3, bb3, lo4, bb4, lo5, bb5):
    outs = ((x3, lo3, bb3), (x4, lo4, bb4), (x5, lo5, bb5))
    for (h, w), (x, lo, bb) in zip(_LEVELS, outs):
        wp = w + 2
        np_ = (h + 2) * (w + 2)
        acc = jnp.zeros((256, np_), dtype=jnp.float32)
        for k in range(9):
            off = (k // 3) * wp + (k % 3)
            acc += jnp.dot(w9[k], x[:, off:off + np_],
                           preferred_element_type=jnp.float32)
        t = jnp.maximum(acc + cb[...], 0.0).astype(jnp.bfloat16)
        lo[...] = jnp.dot(clw[...], t, preferred_element_type=jnp.float32) + clb[...]
        bb[...] = jnp.dot(bbw[...], t, preferred_element_type=jnp.float32) + bbb[...]


def _prep_level(feat, h, w):
    wp = w + 2
    xp = jnp.pad(feat[0], ((0, 0), (1, 1), (1, 1)))
    xp = xp.reshape(256, (h + 2) * wp)
    return jnp.pad(xp, ((0, 0), (wp + 1, wp + 1))).astype(jnp.bfloat16)


def _extract(full, h, w, c):
    return full.reshape(c, h + 2, w + 2)[:, 1:h + 1, 1:w + 1].reshape(1, -1)


def kernel(images, feat_p3, feat_p4, feat_p5, conv_w, conv_b,
           cls_w, cls_b, bbox_w, bbox_b):
    feats = [feat_p3, feat_p4, feat_p5]
    xs = [_prep_level(f, h, w) for f, (h, w) in zip(feats, _LEVELS)]

    w9 = conv_w.transpose(2, 3, 0, 1).reshape(9, 256, 256).astype(jnp.bfloat16)
    cb = conv_b.reshape(256, 1)
    clw = cls_w.reshape(15, 256).astype(jnp.bfloat16)
    clb = cls_b.reshape(15, 1)
    bbw = bbox_w.reshape(60, 256).astype(jnp.bfloat16)
    bbb = bbox_b.reshape(60, 1)

    out_shapes = []
    for h, w in _LEVELS:
        np_ = (h + 2) * (w + 2)
        out_shapes.append(jax.ShapeDtypeStruct((15, np_), jnp.float32))
        out_shapes.append(jax.ShapeDtypeStruct((60, np_), jnp.float32))

    lo3, bb3, lo4, bb4, lo5, bb5 = pl.pallas_call(
        _rpn_head_kernel,
        out_shape=tuple(out_shapes),
    )(xs[0], xs[1], xs[2], w9, cb, clw, clb, bbw, bbb)

    pieces = []
    for (h, w), lo, bb in zip(_LEVELS, (lo3, lo4, lo5), (bb3, bb4, bb5)):
        pieces.append(_extract(lo, h, w, 15))
        pieces.append(_extract(bb, h, w, 60))
    flat = jnp.concatenate(pieces, axis=1)

    anchors = jnp.asarray(
        _anchors_const(images.shape[-2], tuple(h for h, _ in _LEVELS)))
    return (flat, anchors)
